# speedup vs baseline: 1.0275x; 1.0275x over previous
import functools

import jax
import jax.numpy as jnp
from jax import lax
from jax.experimental import pallas as pl
from jax.experimental.pallas import tpu as pltpu
from jax.experimental.pallas import tpu_sc as plsc

D = 768
BATCH = 4
SEQ = 8192
NC = 2
NS = 16
NW = NC * NS
SPW = SEQ // NW
C = 32
NCH = SPW // C
L = 16
VPR = D // L

_mesh = plsc.VectorSubcoreMesh(core_axis_name="c", subcore_axis_name="s")


@functools.partial(
    pl.kernel,
    mesh=_mesh,
    out_type=jax.ShapeDtypeStruct((BATCH * SEQ, D), jnp.float32),
    scratch_types=[
        pltpu.VMEM((C,), jnp.int32),
        pltpu.VMEM((C, D), jnp.float32),
        pltpu.VMEM((C, D), jnp.float32),
        pltpu.SemaphoreType.DMA,
    ],
)
def _embed(xf, tok, pos, out, idxc, tokbuf, posbuf, gsem):
    wid = lax.axis_index("s") * NC + lax.axis_index("c")
    base_s = wid * SPW

    def chunk_body(ch, carry):
        pos0 = base_s + ch * C
        pltpu.sync_copy(pos.at[pl.ds(pos0, C)], posbuf)

        def batch_body(b, carry2):
            row0 = b * SEQ + pos0
            pltpu.sync_copy(xf.at[pl.ds(row0, C)], idxc)
            pltpu.async_copy(tok.at[idxc], tokbuf, gsem).wait()

            def row_body(r, carry3):
                for k in range(VPR):
                    sl = pl.ds(k * L, L)
                    tokbuf[r, sl] = tokbuf[r, sl] + posbuf[r, sl]
                return carry3

            lax.fori_loop(0, C, row_body, 0)
            pltpu.sync_copy(tokbuf, out.at[pl.ds(row0, C)])
            return carry2

        lax.fori_loop(0, BATCH, batch_body, 0)
        return carry

    lax.fori_loop(0, NCH, chunk_body, 0)


def kernel(x, token_emb, pos_emb):
    xf = x.reshape(-1).astype(jnp.int32)
    out = _embed(xf, token_emb, pos_emb)
    return out.reshape(BATCH, SEQ, D)
